# Initial kernel scaffold; baseline (speedup 1.0000x reference)
import jax
import jax.numpy as jnp
from jax.experimental import pallas as pl
from jax.experimental.pallas import tpu as pltpu
from jax.experimental.pallas import tpu_sc as plsc

EMB_DIM = 32
WINDOW = 128


def kernel(token_ids, W):
    B, L = token_ids.shape
    num_idx = B * L
    idx = token_ids.reshape(1, num_idx).astype(jnp.int32)

    mesh = plsc.VectorSubcoreMesh(core_axis_name="core", subcore_axis_name="subcore")

    @pl.kernel(
        out_type=jax.ShapeDtypeStruct((num_idx, EMB_DIM), W.dtype),
        mesh=mesh,
    )
    def gather_kernel(w_hbm, i_hbm, o_hbm):
        def body(i_vmem, o_vmem):
            pltpu.sync_copy(w_hbm.at[i_vmem.at[0]], o_vmem)

        pltpu.emit_pipeline(
            body,
            grid=(num_idx // WINDOW,),
            in_specs=[pl.BlockSpec((1, WINDOW), index_map=lambda i: (0, i))],
            out_specs=[pl.BlockSpec((WINDOW, EMB_DIM), index_map=lambda i: (i, 0))],
            core_axis_name=("core", "subcore"),
            dimension_semantics=(pltpu.PARALLEL,),
        )(i_hbm, o_hbm)

    out = gather_kernel(W, idx)
    return out.reshape(B, L, EMB_DIM)


# speedup vs baseline: 1.0437x; 1.0437x over previous
import jax
import jax.numpy as jnp
from jax.experimental import pallas as pl
from jax.experimental.pallas import tpu as pltpu
from jax.experimental.pallas import tpu_sc as plsc

EMB_DIM = 32
WINDOW = 128


def kernel(token_ids, W):
    B, L = token_ids.shape
    num_idx = B * L
    idx = token_ids.reshape(1, num_idx).astype(jnp.int32)

    mesh = plsc.VectorSubcoreMesh(core_axis_name="core", subcore_axis_name="subcore")

    @pl.kernel(
        out_type=jax.ShapeDtypeStruct((num_idx, EMB_DIM), W.dtype),
        mesh=mesh,
        compiler_params=pltpu.CompilerParams(use_tc_tiling_on_sc=False),
    )
    def gather_kernel(w_hbm, i_hbm, o_hbm):
        def body(i_vmem, o_vmem):
            pltpu.sync_copy(w_hbm.at[i_vmem.at[0]], o_vmem)

        pltpu.emit_pipeline(
            body,
            grid=(num_idx // WINDOW,),
            in_specs=[pl.BlockSpec((1, WINDOW), index_map=lambda i: (0, i))],
            out_specs=[pl.BlockSpec((WINDOW, EMB_DIM), index_map=lambda i: (i, 0))],
            core_axis_name=("core", "subcore"),
            dimension_semantics=(pltpu.PARALLEL,),
        )(i_hbm, o_hbm)

    out = gather_kernel(W, idx)
    return out.reshape(B, L, EMB_DIM)


# speedup vs baseline: 1.7896x; 1.7146x over previous
import jax
import jax.numpy as jnp
from jax import lax
from jax.experimental import pallas as pl
from jax.experimental.pallas import tpu as pltpu
from jax.experimental.pallas import tpu_sc as plsc

EMB_DIM = 32
NUM_CORES = 2
NUM_SUBCORES = 16
NUM_WORKERS = NUM_CORES * NUM_SUBCORES
CHUNK_B = 32


def kernel(token_ids, W):
    B, L = token_ids.shape
    num_idx = B * L
    idx = token_ids.reshape(1, num_idx).astype(jnp.int32)
    chunk = CHUNK_B * L
    b_per_worker = B // NUM_WORKERS
    n_chunks = b_per_worker // CHUNK_B

    mesh = plsc.VectorSubcoreMesh(core_axis_name="core", subcore_axis_name="subcore")

    @pl.kernel(
        out_type=jax.ShapeDtypeStruct((B, L, EMB_DIM), W.dtype),
        mesh=mesh,
        scratch_types=[
            pltpu.VMEM((chunk,), jnp.int32),
            pltpu.VMEM((chunk, EMB_DIM), jnp.float32),
            pltpu.SemaphoreType.DMA,
            pltpu.SemaphoreType.DMA,
        ],
        compiler_params=pltpu.CompilerParams(use_tc_tiling_on_sc=False),
    )
    def gather_kernel(w_hbm, i_hbm, o_hbm, idx_v, rows_v, gsem, wsem):
        wid = lax.axis_index("subcore") * NUM_CORES + lax.axis_index("core")
        wb = wid * b_per_worker

        @pl.loop(0, n_chunks)
        def _(c):
            b0 = wb + c * CHUNK_B
            pltpu.sync_copy(i_hbm.at[0, pl.ds(b0 * L, chunk)], idx_v)
            pltpu.async_copy(w_hbm.at[idx_v], rows_v, gsem).wait()

            @pl.loop(0, CHUNK_B)
            def _(j):
                pltpu.async_copy(rows_v.at[pl.ds(j * L, L)], o_hbm.at[b0 + j], wsem)

            @pl.loop(0, CHUNK_B)
            def _(j):
                pltpu.make_async_copy(
                    rows_v.at[pl.ds(j * L, L)], o_hbm.at[b0 + j], wsem
                ).wait()

    return gather_kernel(W, idx)
